# baseline (speedup 1.0000x reference)
import functools

import jax
import jax.numpy as jnp
from jax import lax
from jax.experimental import pallas as pl
from jax.experimental.pallas import tpu as pltpu
from jax.experimental.pallas import tpu_sc as plsc

B = 16384
NF = 26
VOCAB = 100000
D = 32
L = 50
LP = 56
NO = NF + 1
NC = 2
NS = 16
NW = NC * NS
BPW = B // NW
C = 16
NCHUNK = BPW // C
LANES = 16

SID_N = C * NF
FIDX_N = C * NO
QID_N = C * LP
MAXI = 128


def _descr_slices(total):
    off = 0
    while off < total:
        n = min(MAXI, total - off)
        yield off, n
        off += n


def _sc_body(sid_hbm, qid_hbm, stab_hbm, qtab_hbm, out_hbm,
             sid_v, qid_v, fidx_v, obuf, qrow, dn_v, t0_v,
             idsem, gsem, osem):
    wid = lax.axis_index("s") * NC + lax.axis_index("c")
    base = wid * BPW

    pltpu.sync_copy(qtab_hbm.at[pl.ds(0, 1)], t0_v)
    t00 = t0_v[0, 0:16]
    t01 = t0_v[0, 16:32]
    iota = lax.iota(jnp.int32, LANES)

    def fire_ids(ck, par, qpar):
        b0 = base + ck * C
        pltpu.async_copy(sid_hbm.at[pl.ds(b0 * NF, SID_N)],
                         sid_v.at[par], idsem.at[par])
        pltpu.async_copy(qid_hbm.at[pl.ds(b0 * LP, QID_N)],
                         qid_v.at[qpar], idsem.at[par])

    def drain_ids(par, qpar):
        pltpu.make_async_copy(sid_hbm.at[pl.ds(0, SID_N)],
                              sid_v.at[par], idsem.at[par]).wait()
        pltpu.make_async_copy(qid_hbm.at[pl.ds(0, QID_N)],
                              qid_v.at[qpar], idsem.at[par]).wait()

    def prep_and_fire(ck, par, qpar):
        for s in range(FIDX_N // LANES):
            p = iota + (s * LANES)
            c_idx = p // NO
            i_idx = p - c_idx * NO
            src = jnp.minimum(p - c_idx, SID_N - 1)
            val = plsc.load_gather(sid_v.at[par], [src])
            f = val + i_idx * VOCAB
            f = jnp.where(i_idx < NF, f, jnp.zeros_like(f))
            fidx_v[par, pl.ds(s * LANES, LANES)] = f
        for c in range(C):
            qb = c * LP
            npad = jnp.zeros((LANES,), jnp.int32)
            for j in range(L // LANES):
                q = qid_v[qpar, pl.ds(qb + j * LANES, LANES)]
                npad = npad + plsc.all_reduce_population_count(q == 0)
            qt = qid_v[qpar, pl.ds(qb + LP - LANES, LANES)]
            npad = npad + plsc.all_reduce_population_count(
                (qt == 0) & (iota >= 8))
            npf = npad.astype(jnp.float32)
            dn_v[par, c, 0:16] = npf
        for off, n in _descr_slices(FIDX_N):
            pltpu.async_copy(
                stab_hbm.at[fidx_v.at[par].at[pl.ds(off, n)]],
                obuf.at[par].at[pl.ds(off, n)], gsem.at[par])
        for off, n in _descr_slices(QID_N):
            pltpu.async_copy(
                qtab_hbm.at[qid_v.at[qpar].at[pl.ds(off, n)]],
                qrow.at[par].at[pl.ds(off, n)], gsem.at[par])

    def drain_gathers(par):
        pltpu.make_async_copy(stab_hbm.at[pl.ds(0, FIDX_N)],
                              obuf.at[par], gsem.at[par]).wait()
        pltpu.make_async_copy(qtab_hbm.at[pl.ds(0, QID_N)],
                              qrow.at[par], gsem.at[par]).wait()

    def drain_outwrite(par):
        pltpu.make_async_copy(obuf.at[par],
                              out_hbm.at[pl.ds(0, FIDX_N)],
                              osem.at[par]).wait()

    fire_ids(0, 0, 0)
    fire_ids(1, 1, 1)
    drain_ids(0, 0)
    prep_and_fire(0, 0, 0)

    def chunk_body(g, carry):
        p = g & 1
        pn = 1 - p
        qpn = lax.rem(g + 1, 3)

        @pl.when(g + 1 < NCHUNK)
        def _fire_next():
            drain_ids(pn, qpn)

            @pl.when(g >= 1)
            def _wait_prev_write():
                drain_outwrite(pn)

            prep_and_fire(g + 1, pn, qpn)

            @pl.when(g + 2 < NCHUNK)
            def _fire_ids2():
                fire_ids(g + 2, p, lax.rem(g + 2, 3))

        drain_gathers(p)

        def acc_body(c, carry2):
            qb = c * LP
            acc0 = jnp.zeros((LANES,), jnp.float32)
            acc1 = jnp.zeros((LANES,), jnp.float32)
            for l in range(LP):
                acc0 = acc0 + qrow[p, qb + l, 0:16]
                acc1 = acc1 + qrow[p, qb + l, 16:32]
            npf = dn_v[p, c, 0:16]
            denom = (jnp.float32(LP) - npf) + jnp.float32(1e-16)
            orow = c * NO + NF
            obuf[p, orow, 0:16] = (acc0 - npf * t00) / denom
            obuf[p, orow, 16:32] = (acc1 - npf * t01) / denom
            return carry2

        lax.fori_loop(0, C, acc_body, 0)

        b0 = base + g * C
        pltpu.async_copy(obuf.at[p],
                         out_hbm.at[pl.ds(b0 * NO, FIDX_N)], osem.at[p])
        return carry

    lax.fori_loop(0, NCHUNK, chunk_body, 0)

    drain_outwrite(0)
    drain_outwrite(1)


_sc_kernel = functools.partial(
    pl.kernel,
    out_type=jax.ShapeDtypeStruct((B * NO, D), jnp.float32),
    mesh=plsc.VectorSubcoreMesh(
        core_axis_name="c", subcore_axis_name="s",
        num_cores=NC, num_subcores=NS),
    compiler_params=pltpu.CompilerParams(
        use_tc_tiling_on_sc=False, needs_layout_passes=False),
    scratch_types=[
        pltpu.VMEM((2, SID_N), jnp.int32),
        pltpu.VMEM((3, QID_N), jnp.int32),
        pltpu.VMEM((2, FIDX_N), jnp.int32),
        pltpu.VMEM((2, FIDX_N, D), jnp.float32),
        pltpu.VMEM((2, QID_N, D), jnp.float32),
        pltpu.VMEM((2, C, LANES), jnp.float32),
        pltpu.VMEM((1, D), jnp.float32),
        pltpu.SemaphoreType.DMA((2,)),
        pltpu.SemaphoreType.DMA((2,)),
        pltpu.SemaphoreType.DMA((2,)),
    ],
)(_sc_body)


@jax.jit
def kernel(sparse_ids, seq_ids, sparse_tables, seq_table):
    sid_flat = sparse_ids.reshape(B * NF)
    qid_flat = jnp.pad(seq_ids, ((0, 0), (0, LP - L))).reshape(B * LP)
    stab = sparse_tables.reshape(NF * VOCAB, D)
    out = _sc_kernel(sid_flat, qid_flat, stab, seq_table)
    return out.reshape(B, NO, D)


# speedup vs baseline: 1.0009x; 1.0009x over previous
import functools

import jax
import jax.numpy as jnp
from jax import lax
from jax.experimental import pallas as pl
from jax.experimental.pallas import tpu as pltpu
from jax.experimental.pallas import tpu_sc as plsc

B = 16384
NF = 26
VOCAB = 100000
D = 32
L = 50
LP = 56
NO = NF + 1
NC = 2
NS = 16
NW = NC * NS
BPW = B // NW
C = 16
NCHUNK = BPW // C
LANES = 16

SID_N = C * NF
FIDX_N = C * NO
QID_N = C * LP
MAXI = 1024


def _descr_slices(total):
    off = 0
    while off < total:
        n = min(MAXI, total - off)
        yield off, n
        off += n


def _sc_body(sid_hbm, qid_hbm, stab_hbm, qtab_hbm, out_hbm,
             sid_v, qid_v, fidx_v, obuf, qrow, dn_v, t0_v,
             idsem, gsem, osem):
    wid = lax.axis_index("s") * NC + lax.axis_index("c")
    base = wid * BPW

    pltpu.sync_copy(qtab_hbm.at[pl.ds(0, 1)], t0_v)
    t00 = t0_v[0, 0:16]
    t01 = t0_v[0, 16:32]
    iota = lax.iota(jnp.int32, LANES)

    def fire_ids(ck, par, qpar):
        b0 = base + ck * C
        pltpu.async_copy(sid_hbm.at[pl.ds(b0 * NF, SID_N)],
                         sid_v.at[par], idsem.at[par])
        pltpu.async_copy(qid_hbm.at[pl.ds(b0 * LP, QID_N)],
                         qid_v.at[qpar], idsem.at[par])

    def drain_ids(par, qpar):
        pltpu.make_async_copy(sid_hbm.at[pl.ds(0, SID_N)],
                              sid_v.at[par], idsem.at[par]).wait()
        pltpu.make_async_copy(qid_hbm.at[pl.ds(0, QID_N)],
                              qid_v.at[qpar], idsem.at[par]).wait()

    def prep_and_fire(ck, par, qpar):
        for s in range(FIDX_N // LANES):
            p = iota + (s * LANES)
            c_idx = p // NO
            i_idx = p - c_idx * NO
            src = jnp.minimum(p - c_idx, SID_N - 1)
            val = plsc.load_gather(sid_v.at[par], [src])
            f = val + i_idx * VOCAB
            f = jnp.where(i_idx < NF, f, jnp.zeros_like(f))
            fidx_v[par, pl.ds(s * LANES, LANES)] = f
        for c in range(C):
            qb = c * LP
            npad = jnp.zeros((LANES,), jnp.int32)
            for j in range(L // LANES):
                q = qid_v[qpar, pl.ds(qb + j * LANES, LANES)]
                npad = npad + plsc.all_reduce_population_count(q == 0)
            qt = qid_v[qpar, pl.ds(qb + LP - LANES, LANES)]
            npad = npad + plsc.all_reduce_population_count(
                (qt == 0) & (iota >= 8))
            npf = npad.astype(jnp.float32)
            dn_v[par, c, 0:16] = npf
        for off, n in _descr_slices(FIDX_N):
            pltpu.async_copy(
                stab_hbm.at[fidx_v.at[par].at[pl.ds(off, n)]],
                obuf.at[par].at[pl.ds(off, n)], gsem.at[par])
        for off, n in _descr_slices(QID_N):
            pltpu.async_copy(
                qtab_hbm.at[qid_v.at[qpar].at[pl.ds(off, n)]],
                qrow.at[par].at[pl.ds(off, n)], gsem.at[par])

    def drain_gathers(par):
        pltpu.make_async_copy(stab_hbm.at[pl.ds(0, FIDX_N)],
                              obuf.at[par], gsem.at[par]).wait()
        pltpu.make_async_copy(qtab_hbm.at[pl.ds(0, QID_N)],
                              qrow.at[par], gsem.at[par]).wait()

    def drain_outwrite(par):
        pltpu.make_async_copy(obuf.at[par],
                              out_hbm.at[pl.ds(0, FIDX_N)],
                              osem.at[par]).wait()

    fire_ids(0, 0, 0)
    fire_ids(1, 1, 1)
    drain_ids(0, 0)
    prep_and_fire(0, 0, 0)

    def chunk_body(g, carry):
        p = g & 1
        pn = 1 - p
        qpn = lax.rem(g + 1, 3)

        @pl.when(g + 1 < NCHUNK)
        def _fire_next():
            drain_ids(pn, qpn)

            @pl.when(g >= 1)
            def _wait_prev_write():
                drain_outwrite(pn)

            prep_and_fire(g + 1, pn, qpn)

            @pl.when(g + 2 < NCHUNK)
            def _fire_ids2():
                fire_ids(g + 2, p, lax.rem(g + 2, 3))

        drain_gathers(p)

        def acc_body(c, carry2):
            qb = c * LP
            acc0 = jnp.zeros((LANES,), jnp.float32)
            acc1 = jnp.zeros((LANES,), jnp.float32)
            for l in range(LP):
                acc0 = acc0 + qrow[p, qb + l, 0:16]
                acc1 = acc1 + qrow[p, qb + l, 16:32]
            npf = dn_v[p, c, 0:16]
            denom = (jnp.float32(LP) - npf) + jnp.float32(1e-16)
            orow = c * NO + NF
            obuf[p, orow, 0:16] = (acc0 - npf * t00) / denom
            obuf[p, orow, 16:32] = (acc1 - npf * t01) / denom
            return carry2

        lax.fori_loop(0, C, acc_body, 0)

        b0 = base + g * C
        pltpu.async_copy(obuf.at[p],
                         out_hbm.at[pl.ds(b0 * NO, FIDX_N)], osem.at[p])
        return carry

    lax.fori_loop(0, NCHUNK, chunk_body, 0)

    drain_outwrite(0)
    drain_outwrite(1)


_sc_kernel = functools.partial(
    pl.kernel,
    out_type=jax.ShapeDtypeStruct((B * NO, D), jnp.float32),
    mesh=plsc.VectorSubcoreMesh(
        core_axis_name="c", subcore_axis_name="s",
        num_cores=NC, num_subcores=NS),
    compiler_params=pltpu.CompilerParams(
        use_tc_tiling_on_sc=False, needs_layout_passes=False),
    scratch_types=[
        pltpu.VMEM((2, SID_N), jnp.int32),
        pltpu.VMEM((3, QID_N), jnp.int32),
        pltpu.VMEM((2, FIDX_N), jnp.int32),
        pltpu.VMEM((2, FIDX_N, D), jnp.float32),
        pltpu.VMEM((2, QID_N, D), jnp.float32),
        pltpu.VMEM((2, C, LANES), jnp.float32),
        pltpu.VMEM((1, D), jnp.float32),
        pltpu.SemaphoreType.DMA((2,)),
        pltpu.SemaphoreType.DMA((2,)),
        pltpu.SemaphoreType.DMA((2,)),
    ],
)(_sc_body)


@jax.jit
def kernel(sparse_ids, seq_ids, sparse_tables, seq_table):
    sid_flat = sparse_ids.reshape(B * NF)
    qid_flat = jnp.pad(seq_ids, ((0, 0), (0, LP - L))).reshape(B * LP)
    stab = sparse_tables.reshape(NF * VOCAB, D)
    out = _sc_kernel(sid_flat, qid_flat, stab, seq_table)
    return out.reshape(B, NO, D)


# speedup vs baseline: 1.4856x; 1.4842x over previous
import functools

import jax
import jax.numpy as jnp
from jax import lax
from jax.experimental import pallas as pl
from jax.experimental.pallas import tpu as pltpu
from jax.experimental.pallas import tpu_sc as plsc

B = 16384
NF = 26
VOCAB = 100000
D = 32
L = 50
LP = 56
NO = NF + 1
NC = 2
NS = 16
NW = NC * NS
BPW = B // NW
LANES = 16

CA = 16
NCHA = BPW // CA
QID_N = CA * L
QPD_N = CA * LP

CB = 32
NCHB = BPW // CB
SID_N = CB * NF
FIDX_N = CB * NO


def _seq_body(qid_hbm, qpd_hbm, qtab_hbm, out_hbm,
              qid_v, qpd_v, qrow, dn_v, pbuf, t0_v, idsem, gsem, osem):
    wid = lax.axis_index("s") * NC + lax.axis_index("c")
    base = wid * BPW

    pltpu.sync_copy(qtab_hbm.at[pl.ds(0, 1)], t0_v)
    t00 = t0_v[0, 0:16]
    t01 = t0_v[0, 16:32]
    iota = lax.iota(jnp.int32, LANES)

    def fire_ids(ck, par, qpar):
        b0 = base + ck * CA
        pltpu.async_copy(qid_hbm.at[pl.ds(b0 * L, QID_N)],
                         qid_v.at[qpar], idsem.at[par])
        pltpu.async_copy(qpd_hbm.at[pl.ds(b0 * LP, QPD_N)],
                         qpd_v.at[par], idsem.at[par])

    def drain_ids(par, qpar):
        pltpu.make_async_copy(qid_hbm.at[pl.ds(0, QID_N)],
                              qid_v.at[qpar], idsem.at[par]).wait()
        pltpu.make_async_copy(qpd_hbm.at[pl.ds(0, QPD_N)],
                              qpd_v.at[par], idsem.at[par]).wait()

    def prep_and_fire(ck, par, qpar):
        for c in range(CA):
            qb = c * LP
            npad = jnp.zeros((LANES,), jnp.int32)
            for j in range(L // LANES):
                q = qpd_v[par, pl.ds(qb + j * LANES, LANES)]
                npad = npad + plsc.all_reduce_population_count(q == 0)
            qt = qpd_v[par, pl.ds(qb + LP - LANES, LANES)]
            npad = npad + plsc.all_reduce_population_count(
                (qt == 0) & (iota >= 8))
            dn_v[par, c, 0:16] = npad.astype(jnp.float32)
        pltpu.async_copy(qtab_hbm.at[qid_v.at[qpar]],
                         qrow.at[par], gsem.at[par])

    def drain_gathers(par):
        pltpu.make_async_copy(qtab_hbm.at[pl.ds(0, QID_N)],
                              qrow.at[par], gsem.at[par]).wait()

    def drain_outwrite(par):
        pltpu.make_async_copy(pbuf.at[par],
                              out_hbm.at[pl.ds(0, CA)],
                              osem.at[par]).wait()

    fire_ids(0, 0, 0)
    fire_ids(1, 1, 1)
    drain_ids(0, 0)
    prep_and_fire(0, 0, 0)

    def chunk_body(g, carry):
        p = g & 1
        pn = 1 - p
        qpn = lax.rem(g + 1, 3)

        @pl.when(g + 1 < NCHA)
        def _fire_next():
            drain_ids(pn, qpn)
            prep_and_fire(g + 1, pn, qpn)

            @pl.when(g + 2 < NCHA)
            def _fire_ids2():
                fire_ids(g + 2, p, lax.rem(g + 2, 3))

        drain_gathers(p)

        @pl.when(g >= 2)
        def _wait_prev_write():
            drain_outwrite(p)

        def acc_body(c, carry2):
            qb = c * L
            acc0 = jnp.zeros((LANES,), jnp.float32)
            acc1 = jnp.zeros((LANES,), jnp.float32)
            for l in range(L):
                acc0 = acc0 + qrow[p, qb + l, 0:16]
                acc1 = acc1 + qrow[p, qb + l, 16:32]
            npf = dn_v[p, c, 0:16]
            denom = (jnp.float32(LP) - npf) + jnp.float32(1e-16)
            pbuf[p, c, 0:16] = (acc0 - (npf - 6.0) * t00) / denom
            pbuf[p, c, 16:32] = (acc1 - (npf - 6.0) * t01) / denom
            return carry2

        lax.fori_loop(0, CA, acc_body, 0)

        b0 = base + g * CA
        pltpu.async_copy(pbuf.at[p], out_hbm.at[pl.ds(b0, CA)],
                         osem.at[p])
        return carry

    lax.fori_loop(0, NCHA, chunk_body, 0)

    drain_outwrite(0)
    drain_outwrite(1)


def _sparse_body(sid_hbm, pooled_hbm, stab_hbm, out_hbm,
                 sid_v, fidx_v, obuf, pbuf, idsem, gsem, osem):
    wid = lax.axis_index("s") * NC + lax.axis_index("c")
    base = wid * BPW
    iota = lax.iota(jnp.int32, LANES)

    def fire_ids(ck, par, ppar):
        b0 = base + ck * CB
        pltpu.async_copy(sid_hbm.at[pl.ds(b0 * NF, SID_N)],
                         sid_v.at[par], idsem.at[par])
        pltpu.async_copy(pooled_hbm.at[pl.ds(b0, CB)],
                         pbuf.at[ppar], idsem.at[par])

    def drain_ids(par, ppar):
        pltpu.make_async_copy(sid_hbm.at[pl.ds(0, SID_N)],
                              sid_v.at[par], idsem.at[par]).wait()
        pltpu.make_async_copy(pooled_hbm.at[pl.ds(0, CB)],
                              pbuf.at[ppar], idsem.at[par]).wait()

    def prep_and_fire(ck, par):
        for s in range(FIDX_N // LANES):
            q = iota + (s * LANES)
            c_idx = q // NO
            i_idx = q - c_idx * NO
            src = jnp.minimum(q - c_idx, SID_N - 1)
            val = plsc.load_gather(sid_v.at[par], [src])
            f = val + i_idx * VOCAB
            f = jnp.where(i_idx < NF, f, jnp.zeros_like(f))
            fidx_v[par, pl.ds(s * LANES, LANES)] = f
        pltpu.async_copy(stab_hbm.at[fidx_v.at[par]],
                         obuf.at[par], gsem.at[par])

    def drain_gathers(par):
        pltpu.make_async_copy(stab_hbm.at[pl.ds(0, FIDX_N)],
                              obuf.at[par], gsem.at[par]).wait()

    def drain_outwrite(par):
        pltpu.make_async_copy(obuf.at[par],
                              out_hbm.at[pl.ds(0, FIDX_N)],
                              osem.at[par]).wait()

    fire_ids(0, 0, 0)
    fire_ids(1, 1, 1)
    drain_ids(0, 0)
    prep_and_fire(0, 0)

    def chunk_body(g, carry):
        p = g & 1
        pn = 1 - p

        @pl.when(g + 1 < NCHB)
        def _fire_next():
            drain_ids(pn, lax.rem(g + 1, 3))

            @pl.when(g >= 1)
            def _wait_prev_write():
                drain_outwrite(pn)

            prep_and_fire(g + 1, pn)

            @pl.when(g + 2 < NCHB)
            def _fire_ids2():
                fire_ids(g + 2, p, lax.rem(g + 2, 3))

        drain_gathers(p)

        def merge_body(c, carry2):
            orow = c * NO + NF
            obuf[p, orow, 0:16] = pbuf[lax.rem(g, 3), c, 0:16]
            obuf[p, orow, 16:32] = pbuf[lax.rem(g, 3), c, 16:32]
            return carry2

        lax.fori_loop(0, CB, merge_body, 0)

        b0 = base + g * CB
        pltpu.async_copy(obuf.at[p],
                         out_hbm.at[pl.ds(b0 * NO, FIDX_N)], osem.at[p])
        return carry

    lax.fori_loop(0, NCHB, chunk_body, 0)

    drain_outwrite(0)
    drain_outwrite(1)


_MESH = plsc.VectorSubcoreMesh(
    core_axis_name="c", subcore_axis_name="s",
    num_cores=NC, num_subcores=NS)
_CPARAMS = pltpu.CompilerParams(
    use_tc_tiling_on_sc=False, needs_layout_passes=False)

_seq_kernel = functools.partial(
    pl.kernel,
    out_type=jax.ShapeDtypeStruct((B, D), jnp.float32),
    mesh=_MESH,
    compiler_params=_CPARAMS,
    scratch_types=[
        pltpu.VMEM((3, QID_N), jnp.int32),
        pltpu.VMEM((2, QPD_N), jnp.int32),
        pltpu.VMEM((2, QID_N, D), jnp.float32),
        pltpu.VMEM((2, CA, LANES), jnp.float32),
        pltpu.VMEM((2, CA, D), jnp.float32),
        pltpu.VMEM((1, D), jnp.float32),
        pltpu.SemaphoreType.DMA((2,)),
        pltpu.SemaphoreType.DMA((2,)),
        pltpu.SemaphoreType.DMA((2,)),
    ],
)(_seq_body)

_sparse_kernel = functools.partial(
    pl.kernel,
    out_type=jax.ShapeDtypeStruct((B * NO, D), jnp.float32),
    mesh=_MESH,
    compiler_params=_CPARAMS,
    scratch_types=[
        pltpu.VMEM((2, SID_N), jnp.int32),
        pltpu.VMEM((2, FIDX_N), jnp.int32),
        pltpu.VMEM((2, FIDX_N, D), jnp.float32),
        pltpu.VMEM((3, CB, D), jnp.float32),
        pltpu.SemaphoreType.DMA((2,)),
        pltpu.SemaphoreType.DMA((2,)),
        pltpu.SemaphoreType.DMA((2,)),
    ],
)(_sparse_body)


@jax.jit
def kernel(sparse_ids, seq_ids, sparse_tables, seq_table):
    sid_flat = sparse_ids.reshape(B * NF)
    qid_flat = seq_ids.reshape(B * L)
    qpd_flat = jnp.pad(seq_ids, ((0, 0), (0, LP - L))).reshape(B * LP)
    stab = sparse_tables.reshape(NF * VOCAB, D)
    pooled = _seq_kernel(qid_flat, qpd_flat, seq_table)
    out = _sparse_kernel(sid_flat, pooled, stab)
    return out.reshape(B, NO, D)


# speedup vs baseline: 1.6229x; 1.0924x over previous
import functools

import jax
import jax.numpy as jnp
from jax import lax
from jax.experimental import pallas as pl
from jax.experimental.pallas import tpu as pltpu
from jax.experimental.pallas import tpu_sc as plsc

B = 16384
NF = 26
VOCAB = 100000
D = 32
L = 50
LP = 56
NO = NF + 1
NC = 2
NS = 16
NW = NC * NS
BPW = B // NW
LANES = 16

CA = 16
NCHA = BPW // CA
QID_N = CA * L
QPD_N = CA * LP

CB = 32
NCHB = BPW // CB
SID_N = CB * NF
FIDX_N = CB * NO


def _seq_body(qid_hbm, qpd_hbm, qtab_hbm, out_hbm,
              qid_v, qpd_v, qrow, dn_v, pbuf, t0_v, idsem, gsem, osem):
    wid = lax.axis_index("s") * NC + lax.axis_index("c")
    base = wid * BPW

    pltpu.sync_copy(qtab_hbm.at[pl.ds(0, 1)], t0_v)
    t00 = t0_v[0, 0:16]
    t01 = t0_v[0, 16:32]
    iota = lax.iota(jnp.int32, LANES)

    def fire_ids(ck, par, qpar):
        b0 = base + ck * CA
        pltpu.async_copy(qid_hbm.at[pl.ds(b0 * L, QID_N)],
                         qid_v.at[qpar], idsem.at[par])
        pltpu.async_copy(qpd_hbm.at[pl.ds(b0 * LP, QPD_N)],
                         qpd_v.at[par], idsem.at[par])

    def drain_ids(par, qpar):
        pltpu.make_async_copy(qid_hbm.at[pl.ds(0, QID_N)],
                              qid_v.at[qpar], idsem.at[par]).wait()
        pltpu.make_async_copy(qpd_hbm.at[pl.ds(0, QPD_N)],
                              qpd_v.at[par], idsem.at[par]).wait()

    def prep_and_fire(ck, par, qpar):
        for c in range(CA):
            qb = c * LP
            npad = jnp.zeros((LANES,), jnp.int32)
            for j in range(L // LANES):
                q = qpd_v[par, pl.ds(qb + j * LANES, LANES)]
                npad = npad + plsc.all_reduce_population_count(q == 0)
            qt = qpd_v[par, pl.ds(qb + LP - LANES, LANES)]
            npad = npad + plsc.all_reduce_population_count(
                (qt == 0) & (iota >= 8))
            dn_v[par, c, 0:16] = npad.astype(jnp.float32)
        pltpu.async_copy(qtab_hbm.at[qid_v.at[qpar]],
                         qrow.at[par], gsem.at[par])

    def drain_gathers(par):
        pltpu.make_async_copy(qtab_hbm.at[pl.ds(0, QID_N)],
                              qrow.at[par], gsem.at[par]).wait()

    def drain_outwrite(par):
        pltpu.make_async_copy(pbuf.at[par],
                              out_hbm.at[pl.ds(0, CA)],
                              osem.at[par]).wait()

    fire_ids(0, 0, 0)
    fire_ids(1, 1, 1)
    drain_ids(0, 0)
    prep_and_fire(0, 0, 0)

    def chunk_body(g, carry):
        p = g & 1
        pn = 1 - p
        qpn = lax.rem(g + 1, 3)

        @pl.when(g + 1 < NCHA)
        def _fire_next():
            drain_ids(pn, qpn)
            prep_and_fire(g + 1, pn, qpn)

            @pl.when(g + 2 < NCHA)
            def _fire_ids2():
                fire_ids(g + 2, p, lax.rem(g + 2, 3))

        drain_gathers(p)

        @pl.when(g >= 2)
        def _wait_prev_write():
            drain_outwrite(p)

        def acc_body(c, carry2):
            qb = c * L
            acc0 = jnp.zeros((LANES,), jnp.float32)
            acc1 = jnp.zeros((LANES,), jnp.float32)
            for l in range(L):
                acc0 = acc0 + qrow[p, qb + l, 0:16]
                acc1 = acc1 + qrow[p, qb + l, 16:32]
            npf = dn_v[p, c, 0:16]
            denom = (jnp.float32(LP) - npf) + jnp.float32(1e-16)
            pbuf[p, c, 0:16] = (acc0 - (npf - 6.0) * t00) / denom
            pbuf[p, c, 16:32] = (acc1 - (npf - 6.0) * t01) / denom
            return carry2

        lax.fori_loop(0, CA, acc_body, 0)

        b0 = base + g * CA
        pltpu.async_copy(pbuf.at[p], out_hbm.at[pl.ds(b0, CA)],
                         osem.at[p])
        return carry

    lax.fori_loop(0, NCHA, chunk_body, 0)

    drain_outwrite(0)
    drain_outwrite(1)


def _sparse_body(sid_hbm, pooled_hbm, stab_hbm, out_hbm,
                 sid_v, fidx_v, gbuf, obuf, pbuf, idsem, gsem, osem):
    wid = lax.axis_index("s") * NC + lax.axis_index("c")
    base = wid * BPW
    iota = lax.iota(jnp.int32, LANES)

    def fire_ids(ck, par, ppar):
        b0 = base + ck * CB
        pltpu.async_copy(sid_hbm.at[pl.ds(b0 * NF, SID_N)],
                         sid_v.at[par], idsem.at[par])
        pltpu.async_copy(pooled_hbm.at[pl.ds(b0, CB)],
                         pbuf.at[ppar], idsem.at[par])

    def drain_ids(par, ppar):
        pltpu.make_async_copy(sid_hbm.at[pl.ds(0, SID_N)],
                              sid_v.at[par], idsem.at[par]).wait()
        pltpu.make_async_copy(pooled_hbm.at[pl.ds(0, CB)],
                              pbuf.at[ppar], idsem.at[par]).wait()

    def prep_and_fire(ck, par):
        for s in range(SID_N // LANES):
            q = iota + (s * LANES)
            i_idx = q // CB
            c_idx = q - i_idx * CB
            val = plsc.load_gather(sid_v.at[par], [c_idx * NF + i_idx])
            fidx_v[par, pl.ds(s * LANES, LANES)] = val + i_idx * VOCAB
        pltpu.async_copy(stab_hbm.at[fidx_v.at[par]],
                         gbuf.at[par], gsem.at[par])

    def drain_gathers(par):
        pltpu.make_async_copy(stab_hbm.at[pl.ds(0, SID_N)],
                              gbuf.at[par], gsem.at[par]).wait()

    def drain_outwrite(par):
        pltpu.make_async_copy(obuf.at[par],
                              out_hbm.at[pl.ds(0, FIDX_N)],
                              osem.at[par]).wait()

    fire_ids(0, 0, 0)
    fire_ids(1, 1, 1)
    drain_ids(0, 0)
    prep_and_fire(0, 0)

    def chunk_body(g, carry):
        p = g & 1
        pn = 1 - p

        @pl.when(g + 1 < NCHB)
        def _fire_next():
            drain_ids(pn, lax.rem(g + 1, 3))
            prep_and_fire(g + 1, pn)

            @pl.when(g + 2 < NCHB)
            def _fire_ids2():
                fire_ids(g + 2, p, lax.rem(g + 2, 3))

        drain_gathers(p)

        @pl.when(g >= 2)
        def _wait_prev_write():
            drain_outwrite(p)

        pq = lax.rem(g, 3)

        def merge_body(c, carry2):
            ob = c * NO
            for i in range(NF):
                obuf[p, ob + i, 0:16] = gbuf[p, i * CB + c, 0:16]
                obuf[p, ob + i, 16:32] = gbuf[p, i * CB + c, 16:32]
            obuf[p, ob + NF, 0:16] = pbuf[pq, c, 0:16]
            obuf[p, ob + NF, 16:32] = pbuf[pq, c, 16:32]
            return carry2

        lax.fori_loop(0, CB, merge_body, 0)

        b0 = base + g * CB
        pltpu.async_copy(obuf.at[p],
                         out_hbm.at[pl.ds(b0 * NO, FIDX_N)], osem.at[p])
        return carry

    lax.fori_loop(0, NCHB, chunk_body, 0)

    drain_outwrite(0)
    drain_outwrite(1)


_MESH = plsc.VectorSubcoreMesh(
    core_axis_name="c", subcore_axis_name="s",
    num_cores=NC, num_subcores=NS)
_CPARAMS = pltpu.CompilerParams(
    use_tc_tiling_on_sc=False, needs_layout_passes=False)

_seq_kernel = functools.partial(
    pl.kernel,
    out_type=jax.ShapeDtypeStruct((B, D), jnp.float32),
    mesh=_MESH,
    compiler_params=_CPARAMS,
    scratch_types=[
        pltpu.VMEM((3, QID_N), jnp.int32),
        pltpu.VMEM((2, QPD_N), jnp.int32),
        pltpu.VMEM((2, QID_N, D), jnp.float32),
        pltpu.VMEM((2, CA, LANES), jnp.float32),
        pltpu.VMEM((2, CA, D), jnp.float32),
        pltpu.VMEM((1, D), jnp.float32),
        pltpu.SemaphoreType.DMA((2,)),
        pltpu.SemaphoreType.DMA((2,)),
        pltpu.SemaphoreType.DMA((2,)),
    ],
)(_seq_body)

_sparse_kernel = functools.partial(
    pl.kernel,
    out_type=jax.ShapeDtypeStruct((B * NO, D), jnp.float32),
    mesh=_MESH,
    compiler_params=_CPARAMS,
    scratch_types=[
        pltpu.VMEM((2, SID_N), jnp.int32),
        pltpu.VMEM((2, SID_N), jnp.int32),
        pltpu.VMEM((2, SID_N, D), jnp.float32),
        pltpu.VMEM((2, FIDX_N, D), jnp.float32),
        pltpu.VMEM((3, CB, D), jnp.float32),
        pltpu.SemaphoreType.DMA((2,)),
        pltpu.SemaphoreType.DMA((2,)),
        pltpu.SemaphoreType.DMA((2,)),
    ],
)(_sparse_body)


@jax.jit
def kernel(sparse_ids, seq_ids, sparse_tables, seq_table):
    sid_flat = sparse_ids.reshape(B * NF)
    qid_flat = seq_ids.reshape(B * L)
    qpd_flat = jnp.pad(seq_ids, ((0, 0), (0, LP - L))).reshape(B * LP)
    stab = sparse_tables.reshape(NF * VOCAB, D)
    pooled = _seq_kernel(qid_flat, qpd_flat, seq_table)
    out = _sparse_kernel(sid_flat, pooled, stab)
    return out.reshape(B, NO, D)


# speedup vs baseline: 1.6437x; 1.0129x over previous
import functools

import jax
import jax.numpy as jnp
from jax import lax
from jax.experimental import pallas as pl
from jax.experimental.pallas import tpu as pltpu
from jax.experimental.pallas import tpu_sc as plsc

B = 16384
NF = 26
VOCAB = 100000
D = 32
L = 50
LP = 56
NO = NF + 1
NC = 2
NS = 16
NW = NC * NS
BPW = B // NW
LANES = 16

CA = 16
NCHA = BPW // CA
QID_N = CA * L
QPD_N = CA * LP

CB = 32
NCHB = BPW // CB
SID_N = CB * NF
FIDX_N = CB * NO


def _seq_body(qid_hbm, qpd_hbm, qtab_hbm, out_hbm,
              qid_v, qpd_v, qrow, dn_v, pbuf, t0_v, idsem, gsem, osem):
    wid = lax.axis_index("s") * NC + lax.axis_index("c")
    base = wid * BPW

    pltpu.sync_copy(qtab_hbm.at[pl.ds(0, 1)], t0_v)
    t00 = t0_v[0, 0:16]
    t01 = t0_v[0, 16:32]
    iota = lax.iota(jnp.int32, LANES)

    def fire_ids(ck, par, qpar):
        b0 = base + ck * CA
        pltpu.async_copy(qid_hbm.at[pl.ds(b0 * L, QID_N)],
                         qid_v.at[qpar], idsem.at[par])
        pltpu.async_copy(qpd_hbm.at[pl.ds(b0 * LP, QPD_N)],
                         qpd_v.at[par], idsem.at[par])

    def drain_ids(par, qpar):
        pltpu.make_async_copy(qid_hbm.at[pl.ds(0, QID_N)],
                              qid_v.at[qpar], idsem.at[par]).wait()
        pltpu.make_async_copy(qpd_hbm.at[pl.ds(0, QPD_N)],
                              qpd_v.at[par], idsem.at[par]).wait()

    def prep_and_fire(ck, par, qpar):
        for c in range(CA):
            qb = c * LP
            npad = jnp.zeros((LANES,), jnp.int32)
            for j in range(L // LANES):
                q = qpd_v[par, pl.ds(qb + j * LANES, LANES)]
                npad = npad + plsc.all_reduce_population_count(q == 0)
            qt = qpd_v[par, pl.ds(qb + LP - LANES, LANES)]
            npad = npad + plsc.all_reduce_population_count(
                (qt == 0) & (iota >= 8))
            dn_v[par, c, 0:16] = npad.astype(jnp.float32)
        pltpu.async_copy(qtab_hbm.at[qid_v.at[qpar]],
                         qrow.at[par], gsem.at[par])

    def drain_gathers(par):
        pltpu.make_async_copy(qtab_hbm.at[pl.ds(0, QID_N)],
                              qrow.at[par], gsem.at[par]).wait()

    def drain_outwrite(par):
        pltpu.make_async_copy(pbuf.at[par],
                              out_hbm.at[pl.ds(0, CA)],
                              osem.at[par]).wait()

    fire_ids(0, 0, 0)
    fire_ids(1, 1, 1)
    drain_ids(0, 0)
    prep_and_fire(0, 0, 0)

    def chunk_body(g, carry):
        p = g & 1
        pn = 1 - p
        qpn = lax.rem(g + 1, 3)

        @pl.when(g + 1 < NCHA)
        def _fire_next():
            drain_ids(pn, qpn)
            prep_and_fire(g + 1, pn, qpn)

            @pl.when(g + 2 < NCHA)
            def _fire_ids2():
                fire_ids(g + 2, p, lax.rem(g + 2, 3))

        drain_gathers(p)

        @pl.when(g >= 2)
        def _wait_prev_write():
            drain_outwrite(p)

        def acc_body(c, carry2):
            qb = c * L
            acc0 = jnp.zeros((LANES,), jnp.float32)
            acc1 = jnp.zeros((LANES,), jnp.float32)
            for l in range(L):
                acc0 = acc0 + qrow[p, qb + l, 0:16]
                acc1 = acc1 + qrow[p, qb + l, 16:32]
            npf = dn_v[p, c, 0:16]
            denom = (jnp.float32(LP) - npf) + jnp.float32(1e-16)
            pbuf[p, c, 0:16] = (acc0 - (npf - 6.0) * t00) / denom
            pbuf[p, c, 16:32] = (acc1 - (npf - 6.0) * t01) / denom
            return carry2

        lax.fori_loop(0, CA, acc_body, 0)

        b0 = base + g * CA
        pltpu.async_copy(pbuf.at[p], out_hbm.at[pl.ds(b0, CA)],
                         osem.at[p])
        return carry

    lax.fori_loop(0, NCHA, chunk_body, 0)

    drain_outwrite(0)
    drain_outwrite(1)


def _sparse_body(sid_hbm, pooled_hbm, stab_hbm, out_hbm,
                 sid_v, fidx_v, gbuf, obuf, pbuf, idsem, gsem, osem):
    wid = lax.axis_index("s") * NC + lax.axis_index("c")
    base = wid * BPW
    iota = lax.iota(jnp.int32, LANES)

    def fire_ids(ck, par, ppar):
        b0 = base + ck * CB
        pltpu.async_copy(sid_hbm.at[pl.ds(b0 * NF, SID_N)],
                         sid_v.at[par], idsem.at[par])
        pltpu.async_copy(pooled_hbm.at[pl.ds(b0, CB)],
                         pbuf.at[ppar], idsem.at[par])

    def drain_ids(par, ppar):
        pltpu.make_async_copy(sid_hbm.at[pl.ds(0, SID_N)],
                              sid_v.at[par], idsem.at[par]).wait()
        pltpu.make_async_copy(pooled_hbm.at[pl.ds(0, CB)],
                              pbuf.at[ppar], idsem.at[par]).wait()

    def prep_and_fire(ck, par):
        for s in range(SID_N // LANES):
            q = iota + (s * LANES)
            i_idx = q // CB
            c_idx = q - i_idx * CB
            val = plsc.load_gather(sid_v.at[par], [c_idx * NF + i_idx])
            fidx_v[par, pl.ds(s * LANES, LANES)] = val + i_idx * VOCAB
        pltpu.async_copy(stab_hbm.at[fidx_v.at[par]],
                         gbuf.at[par], gsem.at[par])

    def drain_gathers(par):
        pltpu.make_async_copy(stab_hbm.at[pl.ds(0, SID_N)],
                              gbuf.at[par], gsem.at[par]).wait()

    def drain_outwrite(par):
        pltpu.make_async_copy(obuf.at[par],
                              out_hbm.at[:, pl.ds(0, CB)],
                              osem.at[par]).wait()

    fire_ids(0, 0, 0)
    fire_ids(1, 1, 1)
    drain_ids(0, 0)
    prep_and_fire(0, 0)

    def chunk_body(g, carry):
        p = g & 1
        pn = 1 - p

        @pl.when(g + 1 < NCHB)
        def _fire_next():
            drain_ids(pn, lax.rem(g + 1, 3))
            prep_and_fire(g + 1, pn)

            @pl.when(g + 2 < NCHB)
            def _fire_ids2():
                fire_ids(g + 2, p, lax.rem(g + 2, 3))

        drain_gathers(p)

        @pl.when(g >= 2)
        def _wait_prev_write():
            drain_outwrite(p)

        pq = lax.rem(g, 3)

        def merge_body(c, carry2):
            cvec = c + jnp.zeros((LANES,), jnp.int32)
            for i in range(NF):
                r0 = gbuf[p, i * CB + c, 0:16]
                r1 = gbuf[p, i * CB + c, 16:32]
                plsc.store_scatter(obuf.at[p], [i * D + iota, cvec], r0)
                plsc.store_scatter(obuf.at[p],
                                   [i * D + LANES + iota, cvec], r1)
            plsc.store_scatter(obuf.at[p], [NF * D + iota, cvec],
                               pbuf[pq, c, 0:16])
            plsc.store_scatter(obuf.at[p], [NF * D + LANES + iota, cvec],
                               pbuf[pq, c, 16:32])
            return carry2

        lax.fori_loop(0, CB, merge_body, 0)

        b0 = base + g * CB
        pltpu.async_copy(obuf.at[p],
                         out_hbm.at[:, pl.ds(b0, CB)], osem.at[p])
        return carry

    lax.fori_loop(0, NCHB, chunk_body, 0)

    drain_outwrite(0)
    drain_outwrite(1)


_MESH = plsc.VectorSubcoreMesh(
    core_axis_name="c", subcore_axis_name="s",
    num_cores=NC, num_subcores=NS)
_CPARAMS = pltpu.CompilerParams(
    use_tc_tiling_on_sc=False, needs_layout_passes=False)

_seq_kernel = functools.partial(
    pl.kernel,
    out_type=jax.ShapeDtypeStruct((B, D), jnp.float32),
    mesh=_MESH,
    compiler_params=_CPARAMS,
    scratch_types=[
        pltpu.VMEM((3, QID_N), jnp.int32),
        pltpu.VMEM((2, QPD_N), jnp.int32),
        pltpu.VMEM((2, QID_N, D), jnp.float32),
        pltpu.VMEM((2, CA, LANES), jnp.float32),
        pltpu.VMEM((2, CA, D), jnp.float32),
        pltpu.VMEM((1, D), jnp.float32),
        pltpu.SemaphoreType.DMA((2,)),
        pltpu.SemaphoreType.DMA((2,)),
        pltpu.SemaphoreType.DMA((2,)),
    ],
)(_seq_body)

_sparse_kernel = functools.partial(
    pl.kernel,
    out_type=jax.ShapeDtypeStruct((NO * D, B), jnp.float32),
    mesh=_MESH,
    compiler_params=_CPARAMS,
    scratch_types=[
        pltpu.VMEM((2, SID_N), jnp.int32),
        pltpu.VMEM((2, SID_N), jnp.int32),
        pltpu.VMEM((2, SID_N, D), jnp.float32),
        pltpu.VMEM((2, NO * D, CB), jnp.float32),
        pltpu.VMEM((3, CB, D), jnp.float32),
        pltpu.SemaphoreType.DMA((2,)),
        pltpu.SemaphoreType.DMA((2,)),
        pltpu.SemaphoreType.DMA((2,)),
    ],
)(_sparse_body)


@jax.jit
def kernel(sparse_ids, seq_ids, sparse_tables, seq_table):
    sid_flat = sparse_ids.reshape(B * NF)
    qid_flat = seq_ids.reshape(B * L)
    qpd_flat = jnp.pad(seq_ids, ((0, 0), (0, LP - L))).reshape(B * LP)
    stab = sparse_tables.reshape(NF * VOCAB, D)
    pooled = _seq_kernel(qid_flat, qpd_flat, seq_table)
    out = _sparse_kernel(sid_flat, pooled, stab)
    return out.reshape(NO, D, B).transpose(2, 0, 1)
